# Initial kernel scaffold; baseline (speedup 1.0000x reference)
import jax
import jax.numpy as jnp
from jax.experimental import pallas as pl


def kernel(ssl_content, W, b, codebook):
    raise NotImplementedError("write your pallas kernel here")


# speedup vs baseline: 1.6561x; 1.6561x over previous
import jax
import jax.numpy as jnp
from jax.experimental import pallas as pl


def _vq_kernel(x_ref, wt_ref, b_ref, c_ref, out_ref):
    x = x_ref[0]
    c = c_ref[...]
    zT = jnp.dot(wt_ref[...], x, preferred_element_type=jnp.float32)
    zT = zT + b_ref[...]
    y = jnp.dot(c, zT, preferred_element_type=jnp.float32)
    z_sq = jnp.sum(zT * zT, axis=0, keepdims=True)
    c_sq = jnp.sum(c * c, axis=1, keepdims=True)
    dist = (z_sq - 2.0 * y) + c_sq
    codes = jnp.argmin(dist, axis=0)
    out_ref[...] = codes[None, None, :].astype(jnp.int32)


def kernel(ssl_content, W, b, codebook):
    B, IN_DIM, T = ssl_content.shape
    K, CODE_DIM = codebook.shape
    Tt = 256
    nT = T // Tt
    Wt = W.T
    b2 = b.reshape(CODE_DIM, 1)

    out = pl.pallas_call(
        _vq_kernel,
        grid=(B, nT),
        in_specs=[
            pl.BlockSpec((1, IN_DIM, Tt), lambda i, j: (i, 0, j)),
            pl.BlockSpec((CODE_DIM, IN_DIM), lambda i, j: (0, 0)),
            pl.BlockSpec((CODE_DIM, 1), lambda i, j: (0, 0)),
            pl.BlockSpec((K, CODE_DIM), lambda i, j: (0, 0)),
        ],
        out_specs=pl.BlockSpec((1, 1, Tt), lambda i, j: (i, 0, j)),
        out_shape=jax.ShapeDtypeStruct((B, 1, T), jnp.int32),
    )(ssl_content, Wt, b2, codebook)
    return out.reshape(B, T)
